# baseline (speedup 1.0000x reference)
import functools
import jax
import jax.numpy as jnp
from jax import lax
from jax.experimental import pallas as pl
from jax.experimental.pallas import tpu as pltpu
from jax.experimental.pallas import tpu_sc as plsc

N = 10000
E = 320000
F = 128
G = 64

NC = 2
NS = 16
NW = NC * NS
EPT = E // NW
NPAD = 10240

CHD = 125
NCHD = EPT // CHD

CHS = 50
NCHS = EPT // CHS

_mesh = plsc.VectorSubcoreMesh(core_axis_name="c", subcore_axis_name="s")


@functools.partial(
    pl.kernel,
    out_type=jax.ShapeDtypeStruct((NC, NPAD, F), jnp.float32),
    mesh=_mesh,
    scratch_types=[
        pltpu.VMEM((NCHD, CHD), jnp.int32),
        pltpu.VMEM((CHD, F), jnp.float32),
        pltpu.VMEM((8, F), jnp.float32),
        pltpu.VMEM_SHARED((NPAD, F), jnp.float32),
        pltpu.SemaphoreType.DMA,
        pltpu.SemaphoreType.DMA,
        pltpu.SemaphoreType.DMA,
        pltpu.SemaphoreType.DMA,
    ],
)
def _deg_kernel(rows_hbm, out_hbm, idx_v, ones_v, zb_v, acc_sh,
                sm0, sm1, sm2, sm3):
    c = lax.axis_index("c")
    s = lax.axis_index("s")
    w = s * NC + c
    sems = (sm0, sm1, sm2, sm3)

    def fill(q, _):
        zb_v[q // 8, pl.ds((q % 8) * 16, 16)] = jnp.zeros((16,), jnp.float32)
        return 0
    lax.fori_loop(0, 64, fill, 0)

    def fill1(q, _):
        ones_v[q // 8, pl.ds((q % 8) * 16, 16)] = jnp.ones((16,), jnp.float32)
        return 0
    lax.fori_loop(0, CHD * 8, fill1, 0)

    def zloop(t, _):
        pltpu.sync_copy(zb_v, acc_sh.at[pl.ds(s * 640 + t * 8, 8)])
        return 0
    lax.fori_loop(0, 80, zloop, 0)
    plsc.subcore_barrier()

    pltpu.sync_copy(rows_hbm.at[w], idx_v)

    def sis(j, k):
        pltpu.async_copy(ones_v, acc_sh.at[idx_v.at[j]], sems[k], add=True)

    def swt(j, k):
        pltpu.make_async_copy(ones_v, acc_sh.at[idx_v.at[j]], sems[k]).wait()

    for k in range(4):
        sis(k, k)

    def body(t, _):
        j0 = 4 * t
        for k in range(4):
            swt(j0 + k, k)
            sis(j0 + k + 4, k)
        return 0
    lax.fori_loop(0, (NCHD - 4) // 4, body, 0)
    for k in range(4):
        swt(NCHD - 4 + k, k)

    plsc.subcore_barrier()

    def wloop(t, _):
        pltpu.sync_copy(acc_sh.at[pl.ds(s * 640 + t * 8, 8)], zb_v)
        pltpu.sync_copy(zb_v, out_hbm.at[c, pl.ds(s * 640 + t * 8, 8)])
        return 0
    lax.fori_loop(0, 80, wloop, 0)


@functools.partial(
    pl.kernel,
    out_type=jax.ShapeDtypeStruct((NC, NPAD, F), jnp.float32),
    mesh=_mesh,
    scratch_types=[
        pltpu.VMEM((4, 1, CHS), jnp.int32),
        pltpu.VMEM((4, 1, CHS), jnp.int32),
        pltpu.VMEM((4, CHS, F), jnp.float32),
        pltpu.VMEM((8, F), jnp.float32),
        pltpu.VMEM_SHARED((NPAD, F), jnp.float32),
        pltpu.SemaphoreType.DMA, pltpu.SemaphoreType.DMA,
        pltpu.SemaphoreType.DMA, pltpu.SemaphoreType.DMA,
        pltpu.SemaphoreType.DMA, pltpu.SemaphoreType.DMA,
        pltpu.SemaphoreType.DMA, pltpu.SemaphoreType.DMA,
        pltpu.SemaphoreType.DMA, pltpu.SemaphoreType.DMA,
        pltpu.SemaphoreType.DMA, pltpu.SemaphoreType.DMA,
        pltpu.SemaphoreType.DMA, pltpu.SemaphoreType.DMA,
        pltpu.SemaphoreType.DMA, pltpu.SemaphoreType.DMA,
    ],
)
def _scat_kernel(rows_hbm, cols_hbm, hs_hbm, out_hbm, ridx_v, cidx_v, rows_v,
                 zb_v, acc_sh,
                 sr0, sr1, sr2, sr3, sc0, sc1, sc2, sc3,
                 sg0, sg1, sg2, sg3, ss0, ss1, ss2, ss3):
    c = lax.axis_index("c")
    s = lax.axis_index("s")
    w = s * NC + c
    semr = (sr0, sr1, sr2, sr3)
    semc = (sc0, sc1, sc2, sc3)
    semg = (sg0, sg1, sg2, sg3)
    sems = (ss0, ss1, ss2, ss3)

    def fill(q, _):
        zb_v[q // 8, pl.ds((q % 8) * 16, 16)] = jnp.zeros((16,), jnp.float32)
        return 0
    lax.fori_loop(0, 64, fill, 0)

    def zloop(t, _):
        pltpu.sync_copy(zb_v, acc_sh.at[pl.ds(s * 640 + t * 8, 8)])
        return 0
    lax.fori_loop(0, 80, zloop, 0)
    plsc.subcore_barrier()

    def rld(j, p):
        pltpu.async_copy(rows_hbm.at[w, j], ridx_v.at[p], semr[p])

    def rwt(j, p):
        pltpu.make_async_copy(rows_hbm.at[w, j], ridx_v.at[p], semr[p]).wait()

    def cld(j, p):
        pltpu.async_copy(cols_hbm.at[w, j], cidx_v.at[p], semc[p])

    def cwt(j, p):
        pltpu.make_async_copy(cols_hbm.at[w, j], cidx_v.at[p], semc[p]).wait()

    def gis(j, p):
        pltpu.async_copy(hs_hbm.at[ridx_v.at[p, 0]], rows_v.at[p], semg[p])

    def gwt(j, p):
        pltpu.make_async_copy(hs_hbm.at[ridx_v.at[p, 0]], rows_v.at[p],
                              semg[p]).wait()

    def sis(j, p):
        pltpu.async_copy(rows_v.at[p], acc_sh.at[cidx_v.at[p, 0]], sems[p],
                         add=True)

    def swt(j, p):
        pltpu.make_async_copy(rows_v.at[p], acc_sh.at[cidx_v.at[p, 0]],
                              sems[p]).wait()

    for p in range(4):
        rld(p, p)
    cld(0, 0)
    cld(1, 1)
    rwt(0, 0)
    gis(0, 0)
    rwt(1, 1)
    gis(1, 1)

    def step(j, p, p2, do_swt=True, do_next=True, do_rld=True):
        gwt(j, p)
        cwt(j, p)
        sis(j, p)
        if do_swt:
            swt(j - 2, p2)
        if do_next:
            rwt(j + 2, p2)
            gis(j + 2, p2)
            cld(j + 2, p2)
        if do_rld:
            rld(j + 4, p)

    step(0, 0, 2, do_swt=False)
    step(1, 1, 3, do_swt=False)

    def body(t, _):
        j0 = 4 * t + 2
        for k in range(4):
            j = j0 + k
            p = (2 + k) % 4
            step(j, p, (p + 2) % 4)
        return 0
    lax.fori_loop(0, (NCHS - 8) // 4, body, 0)

    j = NCHS - 6
    for k in range(6):
        p = (j + k) % 4
        step(j + k, p, (p + 2) % 4,
             do_next=(j + k + 2 < NCHS), do_rld=(j + k + 4 < NCHS))
    swt(NCHS - 2, (NCHS - 2) % 4)
    swt(NCHS - 1, (NCHS - 1) % 4)

    plsc.subcore_barrier()

    def wloop(t, _):
        pltpu.sync_copy(acc_sh.at[pl.ds(s * 640 + t * 8, 8)], zb_v)
        pltpu.sync_copy(zb_v, out_hbm.at[c, pl.ds(s * 640 + t * 8, 8)])
        return 0
    lax.fori_loop(0, 80, wloop, 0)


_BR = 1000
_NB = N // _BR


def _dis_of(deg2_block):
    d = deg2_block[0] + deg2_block[1]
    return jnp.where(d > 0.0, lax.rsqrt(jnp.where(d > 0.0, d, 1.0)), 0.0)


def _tc12_body(x_ref, w_ref, b_ref, deg2_ref, h_ref, hs_ref):
    h = jnp.maximum(
        jnp.dot(x_ref[...], w_ref[...], preferred_element_type=jnp.float32)
        + b_ref[...], 0.0)
    h_ref[...] = h
    hs_ref[...] = _dis_of(deg2_ref) * h


def _tc3_body(h_ref, s2_ref, deg2_ref, bat_ref, w0_ref, w1_ref, b_ref,
              out_ref, pool_ref):
    i = pl.program_id(0)
    ssum = s2_ref[0] + s2_ref[1]
    tx1 = -(_dis_of(deg2_ref) * ssum)
    o = jnp.maximum(
        jnp.dot(h_ref[...], w0_ref[...], preferred_element_type=jnp.float32)
        + jnp.dot(tx1, w1_ref[...], preferred_element_type=jnp.float32)
        + b_ref[...], 0.0)
    out_ref[...] = o
    gids = lax.broadcasted_iota(jnp.int32, (_BR, G), 1).astype(jnp.float32)
    oh = jnp.where(bat_ref[...] == gids, 1.0, 0.0)
    p = lax.dot_general(oh, o, (((0,), (0,)), ((), ())),
                        preferred_element_type=jnp.float32)

    @pl.when(i == 0)
    def _():
        pool_ref[...] = p

    @pl.when(i > 0)
    def _():
        pool_ref[...] = pool_ref[...] + p


def kernel(x2, edge_index2, batch, lin0_W, lin0_b, cheb_W0, cheb_W1, cheb_b):
    ei = edge_index2.astype(jnp.int32)
    rows_deg = ei[0].reshape(NW, NCHD, CHD)
    rows_scat = ei[0].reshape(NW, NCHS, 1, CHS)
    cols_scat = ei[1].reshape(NW, NCHS, 1, CHS)

    deg2 = _deg_kernel(rows_deg)

    h, hs = pl.pallas_call(
        _tc12_body,
        grid=(_NB,),
        in_specs=[
            pl.BlockSpec((_BR, F), lambda i: (i, 0)),
            pl.BlockSpec((F, F), lambda i: (0, 0)),
            pl.BlockSpec((1, F), lambda i: (0, 0)),
            pl.BlockSpec((2, _BR, F), lambda i: (0, i, 0)),
        ],
        out_specs=[
            pl.BlockSpec((_BR, F), lambda i: (i, 0)),
            pl.BlockSpec((_BR, F), lambda i: (i, 0)),
        ],
        out_shape=[
            jax.ShapeDtypeStruct((N, F), jnp.float32),
            jax.ShapeDtypeStruct((N, F), jnp.float32),
        ],
    )(x2, lin0_W, lin0_b.reshape(1, F), deg2)

    s2 = _scat_kernel(rows_scat, cols_scat, hs)

    bat_col = batch.astype(jnp.float32).reshape(N, 1)

    out, pooled = pl.pallas_call(
        _tc3_body,
        grid=(_NB,),
        in_specs=[
            pl.BlockSpec((_BR, F), lambda i: (i, 0)),
            pl.BlockSpec((2, _BR, F), lambda i: (0, i, 0)),
            pl.BlockSpec((2, _BR, F), lambda i: (0, i, 0)),
            pl.BlockSpec((_BR, 1), lambda i: (i, 0)),
            pl.BlockSpec((F, F), lambda i: (0, 0)),
            pl.BlockSpec((F, F), lambda i: (0, 0)),
            pl.BlockSpec((1, F), lambda i: (0, 0)),
        ],
        out_specs=[
            pl.BlockSpec((_BR, F), lambda i: (i, 0)),
            pl.BlockSpec((G, F), lambda i: (0, 0)),
        ],
        out_shape=[
            jax.ShapeDtypeStruct((N, F), jnp.float32),
            jax.ShapeDtypeStruct((G, F), jnp.float32),
        ],
    )(h, s2, deg2, bat_col, cheb_W0, cheb_W1, cheb_b.reshape(1, F))

    return (pooled, out)
